# baseline (speedup 1.0000x reference)
import functools

import jax
import jax.numpy as jnp
from jax import lax
from jax.experimental import pallas as pl
from jax.experimental.pallas import tpu as pltpu
from jax.experimental.pallas import tpu_sc as plsc

N = 10000
NPAD = 10240
E = 320000
G = 64

NC = 2
NS = 16
NW = NC * NS
EPW = E // NW
K = 128
EPWP = 10240
NCH = EPWP // K
NBUF = 4
NIB = 8
RPT = NPAD // NS

EPT = E // NS
EPTP = 20480
NCHS = EPTP // K
HW = 64

BR = 1024
NB = NPAD // BR


def _make_sc_segsum(W: int, split: bool):
    mesh = plsc.VectorSubcoreMesh(core_axis_name="c", subcore_axis_name="s")
    n_ch = NCHS if split else NCH

    @functools.partial(
        pl.kernel,
        mesh=mesh,
        compiler_params=pltpu.CompilerParams(use_tc_tiling_on_sc=False),
        out_type=jax.ShapeDtypeStruct((NC, NPAD, W), jnp.float32),
        scratch_types=[
            pltpu.VMEM((NIB, 2, K), jnp.int32),
            pltpu.VMEM((NIB, K), jnp.float32),
            pltpu.VMEM((NBUF, K, W), jnp.float32),
            pltpu.VMEM_SHARED((NPAD, W), jnp.float32),
            pltpu.VMEM_SHARED((NPAD, W), jnp.float32),
            pltpu.SemaphoreType.DMA((NIB,)),
            pltpu.SemaphoreType.DMA((NIB,)),
            pltpu.SemaphoreType.DMA((NBUF,)),
            pltpu.SemaphoreType.DMA((NBUF,)),
        ],
    )
    def seg_kernel(h_hbm, pk_hbm, ew_hbm, out_hbm,
                   idx_v, ewr_v, rows_v, hsh, acc, isem, esem, gsem, ssem):
        c = lax.axis_index("c")
        s = lax.axis_index("s")
        w = s if split else s * NC + c

        hsrc = h_hbm.at[c] if split else h_hbm
        pltpu.sync_copy(hsrc.at[pl.ds(s * RPT, RPT)],
                        hsh.at[pl.ds(s * RPT, RPT)])

        @plsc.parallel_loop(0, K, 1, unroll=4)
        def zrow(r):
            for wi in range(W // 16):
                rows_v[0, r, pl.ds(wi * 16, 16)] = jnp.zeros((16,),
                                                             jnp.float32)
        for j in range(RPT // K):
            pltpu.sync_copy(rows_v.at[0], acc.at[pl.ds(s * RPT + j * K, K)])
        plsc.subcore_barrier()

        def idx_start(ci, ib):
            pltpu.make_async_copy(pk_hbm.at[w, ci], idx_v.at[ib],
                                  isem.at[ib]).start()
            pltpu.make_async_copy(ew_hbm.at[w, ci], ewr_v.at[ib],
                                  esem.at[ib]).start()

        def idx_wait(ci, ib):
            pltpu.make_async_copy(pk_hbm.at[w, ci], idx_v.at[ib],
                                  isem.at[ib]).wait()
            pltpu.make_async_copy(ew_hbm.at[w, ci], ewr_v.at[ib],
                                  esem.at[ib]).wait()

        def gather_start(b, ib):
            pltpu.make_async_copy(hsh.at[idx_v.at[ib, 0]],
                                  rows_v.at[b], gsem.at[b]).start()

        def gather_wait(b, ib):
            pltpu.make_async_copy(hsh.at[idx_v.at[ib, 0]],
                                  rows_v.at[b], gsem.at[b]).wait()

        def scatter_start(b, ib):
            pltpu.make_async_copy(rows_v.at[b], acc.at[idx_v.at[ib, 1]],
                                  ssem.at[b]).start(add=True)

        def scatter_wait(b, ib):
            pltpu.make_async_copy(rows_v.at[b], acc.at[idx_v.at[ib, 1]],
                                  ssem.at[b]).wait()

        idx_start(0, 0)
        idx_start(1, 1)
        idx_start(2, 2)
        idx_wait(0, 0)
        gather_start(0, 0)
        idx_wait(1, 1)
        gather_start(1, 1)

        def outer(ii, _):
            for slot in range(NIB):
                ci = ii * NIB + slot
                b = slot % NBUF
                ib = slot

                @pl.when(ci + 3 < n_ch)
                def _():
                    idx_start(ci + 3, (ib + 3) % NIB)

                @pl.when(ci >= 2)
                def _():
                    scatter_wait((b + 2) % NBUF, (ib + 6) % NIB)

                @pl.when(ci + 2 < n_ch)
                def _():
                    idx_wait(ci + 2, (ib + 2) % NIB)
                    gather_start((b + 2) % NBUF, (ib + 2) % NIB)

                gather_wait(b, ib)

                @plsc.parallel_loop(0, K // 16, 1, unroll=2)
                def scale(q):
                    ew16 = ewr_v[ib, pl.ds(q * 16, 16)]
                    for j in range(16):
                        sval = ew16[j]
                        e = q * 16 + j
                        for wi in range(W // 16):
                            rows_v[b, e, pl.ds(wi * 16, 16)] = (
                                rows_v[b, e, pl.ds(wi * 16, 16)] * sval)
                scatter_start(b, ib)
            return 0
        lax.fori_loop(0, n_ch // NIB, outer, 0)
        scatter_wait((n_ch - 2) % NBUF, (n_ch - 2) % NIB)
        scatter_wait((n_ch - 1) % NBUF, (n_ch - 1) % NIB)
        plsc.subcore_barrier()

        pltpu.sync_copy(acc.at[pl.ds(s * RPT, RPT)],
                        out_hbm.at[c, pl.ds(s * RPT, RPT)])

    return seg_kernel


_SC_SEGSUM = {W: _make_sc_segsum(W, split=False) for W in (16, 64)}
_SC_SPLIT = _make_sc_segsum(HW, split=True)


def _make_tc_layer(Wa, din, dout, apply_wr, agg_cat=False, hin_split=False,
                   hout_split=False, wnext_dim=None, ynext_split=False):
    def body(*refs):
        refs = list(refs)
        agg_ref = refs.pop(0)
        h_ref = refs.pop(0)
        wr_ref = refs.pop(0) if apply_wr else None
        wo_ref = refs.pop(0)
        b_ref = refs.pop(0)
        wy_ref = refs.pop(0) if wnext_dim is not None else None
        if agg_cat:
            aggs = jnp.concatenate([agg_ref[0], agg_ref[1]], axis=1)
        else:
            aggs = agg_ref[0] + agg_ref[1]
        if apply_wr:
            t = jnp.dot(aggs, wr_ref[...], preferred_element_type=jnp.float32)
        else:
            t = aggs
        if hin_split:
            h = jnp.concatenate([h_ref[0], h_ref[1]], axis=1)
        else:
            h = h_ref[...]
        hnew = (t + b_ref[...]) + jnp.dot(h, wo_ref[...],
                                          preferred_element_type=jnp.float32)
        hnew = jnp.maximum(hnew, 0.0)
        hout_ref = refs.pop(0)
        if hout_split:
            hout_ref[0] = hnew[:, :dout // 2]
            hout_ref[1] = hnew[:, dout // 2:]
        else:
            hout_ref[...] = hnew
        if wnext_dim is not None:
            y = jnp.dot(hnew, wy_ref[...], preferred_element_type=jnp.float32)
            yout_ref = refs.pop(0)
            if ynext_split:
                yout_ref[0] = y[:, :wnext_dim // 2]
                yout_ref[1] = y[:, wnext_dim // 2:]
            else:
                yout_ref[...] = y

    agg_w = Wa // 2 if agg_cat else Wa
    in_specs = [pl.BlockSpec((NC, BR, agg_w), lambda i: (0, i, 0))]
    if hin_split:
        in_specs.append(pl.BlockSpec((NC, BR, din // 2), lambda i: (0, i, 0)))
    else:
        in_specs.append(pl.BlockSpec((BR, din), lambda i: (i, 0)))
    if apply_wr:
        in_specs.append(pl.BlockSpec((Wa, dout), lambda i: (0, 0)))
    in_specs.append(pl.BlockSpec((din, dout), lambda i: (0, 0)))
    in_specs.append(pl.BlockSpec((1, dout), lambda i: (0, 0)))
    if wnext_dim is not None:
        in_specs.append(pl.BlockSpec((dout, wnext_dim), lambda i: (0, 0)))
    out_specs, out_shape = [], []
    if hout_split:
        out_specs.append(pl.BlockSpec((NC, BR, dout // 2), lambda i: (0, i, 0)))
        out_shape.append(jax.ShapeDtypeStruct((NC, NPAD, dout // 2),
                                              jnp.float32))
    else:
        out_specs.append(pl.BlockSpec((BR, dout), lambda i: (i, 0)))
        out_shape.append(jax.ShapeDtypeStruct((NPAD, dout), jnp.float32))
    if wnext_dim is not None:
        if ynext_split:
            out_specs.append(
                pl.BlockSpec((NC, BR, wnext_dim // 2), lambda i: (0, i, 0)))
            out_shape.append(
                jax.ShapeDtypeStruct((NC, NPAD, wnext_dim // 2), jnp.float32))
        else:
            out_specs.append(pl.BlockSpec((BR, wnext_dim), lambda i: (i, 0)))
            out_shape.append(jax.ShapeDtypeStruct((NPAD, wnext_dim),
                                                  jnp.float32))

    f = pl.pallas_call(
        body,
        grid=(NB,),
        in_specs=in_specs,
        out_specs=out_specs if len(out_specs) > 1 else out_specs[0],
        out_shape=out_shape if len(out_shape) > 1 else out_shape[0],
    )
    return f


def _make_tc_pool(Wa, din, dout):
    def body(agg_ref, h_ref, wo_ref, b_ref, batch_ref, sums_ref, cnt_ref):
        i = pl.program_id(0)
        hnew = (agg_ref[0] + agg_ref[1] + b_ref[...]) + jnp.dot(
            h_ref[...], wo_ref[...], preferred_element_type=jnp.float32)
        hnew = jnp.maximum(hnew, 0.0)
        bvec = batch_ref[0, 0]
        oh = (bvec[:, None] == lax.broadcasted_iota(jnp.int32, (1, G), 1)
              ).astype(jnp.float32)
        bs = lax.dot_general(oh, hnew, (((0,), (0,)), ((), ())),
                             preferred_element_type=jnp.float32)
        bc = lax.dot_general(oh, jnp.ones((BR, dout), jnp.float32),
                             (((0,), (0,)), ((), ())),
                             preferred_element_type=jnp.float32)

        @pl.when(i == 0)
        def _():
            sums_ref[...] = bs
            cnt_ref[...] = bc

        @pl.when(i > 0)
        def _():
            sums_ref[...] += bs
            cnt_ref[...] += bc

    return pl.pallas_call(
        body,
        grid=(NB,),
        in_specs=[
            pl.BlockSpec((NC, BR, Wa), lambda i: (0, i, 0)),
            pl.BlockSpec((BR, din), lambda i: (i, 0)),
            pl.BlockSpec((din, dout), lambda i: (0, 0)),
            pl.BlockSpec((1, dout), lambda i: (0, 0)),
            pl.BlockSpec((1, 1, BR), lambda i: (i, 0, 0)),
        ],
        out_specs=[
            pl.BlockSpec((G, dout), lambda i: (0, 0)),
            pl.BlockSpec((G, dout), lambda i: (0, 0)),
        ],
        out_shape=[
            jax.ShapeDtypeStruct((G, dout), jnp.float32),
            jax.ShapeDtypeStruct((G, dout), jnp.float32),
        ],
    )


def _make_tc_mlp():
    def body(sums_ref, cnt_ref, w0, b0, w1, b1, w2, b2, out_ref):
        h = sums_ref[...] / jnp.maximum(cnt_ref[...], 1.0)
        h = jnp.maximum(jnp.dot(h, w0[...], preferred_element_type=jnp.float32)
                        + b0[...], 0.0)
        h = jnp.maximum(jnp.dot(h, w1[...], preferred_element_type=jnp.float32)
                        + b1[...], 0.0)
        out_ref[...] = jnp.dot(h, w2[...],
                               preferred_element_type=jnp.float32) + b2[...]

    return pl.pallas_call(
        body,
        out_shape=jax.ShapeDtypeStruct((G, 1), jnp.float32),
    )


_TC0 = _make_tc_layer(16, 16, 64, apply_wr=True)
_TC1 = _make_tc_layer(64, 64, 128, apply_wr=True, hout_split=True)
_TC2 = _make_tc_layer(128, 128, 256, apply_wr=True, agg_cat=True,
                      hin_split=True, wnext_dim=128, ynext_split=True)
_TC3 = _make_tc_layer(128, 256, 128, apply_wr=False, agg_cat=True,
                      wnext_dim=64)
_TC4 = _make_tc_pool(64, 128, 64)
_TCMLP = _make_tc_mlp()


def kernel(x, edge_index, edge_attr, batch,
           Wrel0, brel0, Wroot0,
           Wrel1, brel1, Wroot1,
           Wrel2, brel2, Wroot2,
           Wrel3, brel3, Wroot3,
           Wrel4, brel4, Wroot4,
           Wm0, bm0, Wm1, bm1, Wm2, bm2):
    pad = ((0, 0), (0, EPWP - EPW))
    pkf = jnp.stack([
        jnp.pad(edge_index[0].reshape(NW, EPW), pad).reshape(NW, NCH, K),
        jnp.pad(edge_index[1].reshape(NW, EPW), pad).reshape(NW, NCH, K),
    ], axis=2)
    ewf = jnp.pad(edge_attr.reshape(NW, EPW), pad).reshape(NW, NCH, K)

    padt = ((0, 0), (0, EPTP - EPT))
    pk = jnp.stack([
        jnp.pad(edge_index[0].reshape(NS, EPT), padt).reshape(NS, NCHS, K),
        jnp.pad(edge_index[1].reshape(NS, EPT), padt).reshape(NS, NCHS, K),
    ], axis=2)
    ewt = jnp.pad(edge_attr.reshape(NS, EPT), padt).reshape(NS, NCHS, K)

    xpad = jnp.zeros((NPAD, 16), jnp.float32).at[:N, :5].set(x)
    batch_pad = jnp.full((NPAD,), G, jnp.int32).at[:N].set(batch)
    batch3 = batch_pad.reshape(NB, 1, BR)

    Wr0p = jnp.zeros((16, 64), jnp.float32).at[:5].set(Wrel0)
    Wo0p = jnp.zeros((16, 64), jnp.float32).at[:5].set(Wroot0)

    b0 = brel0.reshape(1, -1)
    b1 = brel1.reshape(1, -1)
    b2 = brel2.reshape(1, -1)
    b3 = brel3.reshape(1, -1)
    b4 = brel4.reshape(1, -1)

    agg0 = _SC_SEGSUM[16](xpad, pkf, ewf)
    h1 = _TC0(agg0, xpad, Wr0p, Wo0p, b0)
    agg1 = _SC_SEGSUM[64](h1, pkf, ewf)
    h2s = _TC1(agg1, h1, Wrel1, Wroot1, b1)
    agg2 = _SC_SPLIT(h2s, pk, ewt)
    h3, y3s = _TC2(agg2, h2s, Wrel2, Wroot2, b2, Wrel3)
    agg3 = _SC_SPLIT(y3s, pk, ewt)
    h4, y4 = _TC3(agg3, h3, Wroot3, b3, Wrel4)
    agg4 = _SC_SEGSUM[64](y4, pkf, ewf)
    sums, cnt = _TC4(agg4, h4, Wroot4, b4, batch3)
    out = _TCMLP(sums, cnt, Wm0, bm0.reshape(1, -1),
                 Wm1, bm1.reshape(1, -1), Wm2, bm2.reshape(1, -1))
    return out


# speedup vs baseline: 1.0186x; 1.0186x over previous
import functools

import jax
import jax.numpy as jnp
from jax import lax
from jax.experimental import pallas as pl
from jax.experimental.pallas import tpu as pltpu
from jax.experimental.pallas import tpu_sc as plsc

N = 10000
NPAD = 10240
E = 320000
G = 64

NC = 2
NS = 16
NW = NC * NS
EPW = E // NW
K = 128
EPWP = 10240
NCH = EPWP // K
NBUF = 4
NIB = 8
RPT = NPAD // NS

EPT = E // NS
EPTP = 20480
NCHS = EPTP // K
HW = 64

BR = 1024
NB = NPAD // BR


def _make_sc_segsum(W: int, split: bool):
    mesh = plsc.VectorSubcoreMesh(core_axis_name="c", subcore_axis_name="s")
    n_ch = NCHS if split else NCH

    @functools.partial(
        pl.kernel,
        mesh=mesh,
        compiler_params=pltpu.CompilerParams(use_tc_tiling_on_sc=False),
        out_type=jax.ShapeDtypeStruct((NC, NPAD, W), jnp.float32),
        scratch_types=[
            pltpu.VMEM((NIB, 2, K), jnp.int32),
            pltpu.VMEM((NIB, K), jnp.float32),
            pltpu.VMEM((NBUF, K, W), jnp.float32),
            pltpu.VMEM_SHARED((NPAD, W), jnp.float32),
            pltpu.VMEM_SHARED((NPAD, W), jnp.float32),
            pltpu.SemaphoreType.DMA((NIB,)),
            pltpu.SemaphoreType.DMA((NIB,)),
            pltpu.SemaphoreType.DMA((NBUF,)),
            pltpu.SemaphoreType.DMA((NBUF,)),
        ],
    )
    def seg_kernel(h_hbm, pk_hbm, ew_hbm, out_hbm,
                   idx_v, ewr_v, rows_v, hsh, acc, isem, esem, gsem, ssem):
        c = lax.axis_index("c")
        s = lax.axis_index("s")
        w = s if split else s * NC + c

        hsrc = h_hbm.at[c] if split else h_hbm
        hstage = pltpu.make_async_copy(hsrc.at[pl.ds(s * RPT, RPT)],
                                       hsh.at[pl.ds(s * RPT, RPT)],
                                       gsem.at[0])
        hstage.start()

        @plsc.parallel_loop(0, K, 1, unroll=4)
        def zrow(r):
            for wi in range(W // 16):
                rows_v[0, r, pl.ds(wi * 16, 16)] = jnp.zeros((16,),
                                                             jnp.float32)
        for j in range(RPT // K):
            pltpu.sync_copy(rows_v.at[0], acc.at[pl.ds(s * RPT + j * K, K)])
        hstage.wait()
        plsc.subcore_barrier()

        def idx_start(ci, ib):
            pltpu.make_async_copy(pk_hbm.at[w, ci], idx_v.at[ib],
                                  isem.at[ib]).start()
            pltpu.make_async_copy(ew_hbm.at[w, ci], ewr_v.at[ib],
                                  esem.at[ib]).start()

        def idx_wait(ci, ib):
            pltpu.make_async_copy(pk_hbm.at[w, ci], idx_v.at[ib],
                                  isem.at[ib]).wait()
            pltpu.make_async_copy(ew_hbm.at[w, ci], ewr_v.at[ib],
                                  esem.at[ib]).wait()

        def gather_start(b, ib):
            pltpu.make_async_copy(hsh.at[idx_v.at[ib, 0]],
                                  rows_v.at[b], gsem.at[b]).start()

        def gather_wait(b, ib):
            pltpu.make_async_copy(hsh.at[idx_v.at[ib, 0]],
                                  rows_v.at[b], gsem.at[b]).wait()

        def scatter_start(b, ib):
            pltpu.make_async_copy(rows_v.at[b], acc.at[idx_v.at[ib, 1]],
                                  ssem.at[b]).start(add=True)

        def scatter_wait(b, ib):
            pltpu.make_async_copy(rows_v.at[b], acc.at[idx_v.at[ib, 1]],
                                  ssem.at[b]).wait()

        idx_start(0, 0)
        idx_start(1, 1)
        idx_start(2, 2)
        idx_wait(0, 0)
        gather_start(0, 0)
        idx_wait(1, 1)
        gather_start(1, 1)

        def outer(ii, _):
            for slot in range(NIB):
                ci = ii * NIB + slot
                b = slot % NBUF
                ib = slot

                @pl.when(ci + 3 < n_ch)
                def _():
                    idx_start(ci + 3, (ib + 3) % NIB)

                @pl.when(ci >= 2)
                def _():
                    scatter_wait((b + 2) % NBUF, (ib + 6) % NIB)

                @pl.when(ci + 2 < n_ch)
                def _():
                    idx_wait(ci + 2, (ib + 2) % NIB)
                    gather_start((b + 2) % NBUF, (ib + 2) % NIB)

                gather_wait(b, ib)

                @plsc.parallel_loop(0, K // 16, 1, unroll=4)
                def scale(q):
                    ew16 = ewr_v[ib, pl.ds(q * 16, 16)]
                    for j in range(16):
                        sval = ew16[j]
                        e = q * 16 + j
                        for wi in range(W // 16):
                            rows_v[b, e, pl.ds(wi * 16, 16)] = (
                                rows_v[b, e, pl.ds(wi * 16, 16)] * sval)
                scatter_start(b, ib)
            return 0
        lax.fori_loop(0, n_ch // NIB, outer, 0)
        scatter_wait((n_ch - 2) % NBUF, (n_ch - 2) % NIB)
        scatter_wait((n_ch - 1) % NBUF, (n_ch - 1) % NIB)
        plsc.subcore_barrier()

        pltpu.sync_copy(acc.at[pl.ds(s * RPT, RPT)],
                        out_hbm.at[c, pl.ds(s * RPT, RPT)])

    return seg_kernel


_SC_SEGSUM = {W: _make_sc_segsum(W, split=False) for W in (16, 64)}
_SC_SPLIT = _make_sc_segsum(HW, split=True)


def _make_tc_layer(Wa, din, dout, apply_wr, agg_cat=False, hin_split=False,
                   hout_split=False, wnext_dim=None, ynext_split=False):
    def body(*refs):
        refs = list(refs)
        agg_ref = refs.pop(0)
        h_ref = refs.pop(0)
        wr_ref = refs.pop(0) if apply_wr else None
        wo_ref = refs.pop(0)
        b_ref = refs.pop(0)
        wy_ref = refs.pop(0) if wnext_dim is not None else None
        if agg_cat:
            aggs = jnp.concatenate([agg_ref[0], agg_ref[1]], axis=1)
        else:
            aggs = agg_ref[0] + agg_ref[1]
        if apply_wr:
            t = jnp.dot(aggs, wr_ref[...], preferred_element_type=jnp.float32)
        else:
            t = aggs
        if hin_split:
            h = jnp.concatenate([h_ref[0], h_ref[1]], axis=1)
        else:
            h = h_ref[...]
        hnew = (t + b_ref[...]) + jnp.dot(h, wo_ref[...],
                                          preferred_element_type=jnp.float32)
        hnew = jnp.maximum(hnew, 0.0)
        hout_ref = refs.pop(0)
        if hout_split:
            hout_ref[0] = hnew[:, :dout // 2]
            hout_ref[1] = hnew[:, dout // 2:]
        else:
            hout_ref[...] = hnew
        if wnext_dim is not None:
            y = jnp.dot(hnew, wy_ref[...], preferred_element_type=jnp.float32)
            yout_ref = refs.pop(0)
            if ynext_split:
                yout_ref[0] = y[:, :wnext_dim // 2]
                yout_ref[1] = y[:, wnext_dim // 2:]
            else:
                yout_ref[...] = y

    agg_w = Wa // 2 if agg_cat else Wa
    in_specs = [pl.BlockSpec((NC, BR, agg_w), lambda i: (0, i, 0))]
    if hin_split:
        in_specs.append(pl.BlockSpec((NC, BR, din // 2), lambda i: (0, i, 0)))
    else:
        in_specs.append(pl.BlockSpec((BR, din), lambda i: (i, 0)))
    if apply_wr:
        in_specs.append(pl.BlockSpec((Wa, dout), lambda i: (0, 0)))
    in_specs.append(pl.BlockSpec((din, dout), lambda i: (0, 0)))
    in_specs.append(pl.BlockSpec((1, dout), lambda i: (0, 0)))
    if wnext_dim is not None:
        in_specs.append(pl.BlockSpec((dout, wnext_dim), lambda i: (0, 0)))
    out_specs, out_shape = [], []
    if hout_split:
        out_specs.append(pl.BlockSpec((NC, BR, dout // 2), lambda i: (0, i, 0)))
        out_shape.append(jax.ShapeDtypeStruct((NC, NPAD, dout // 2),
                                              jnp.float32))
    else:
        out_specs.append(pl.BlockSpec((BR, dout), lambda i: (i, 0)))
        out_shape.append(jax.ShapeDtypeStruct((NPAD, dout), jnp.float32))
    if wnext_dim is not None:
        if ynext_split:
            out_specs.append(
                pl.BlockSpec((NC, BR, wnext_dim // 2), lambda i: (0, i, 0)))
            out_shape.append(
                jax.ShapeDtypeStruct((NC, NPAD, wnext_dim // 2), jnp.float32))
        else:
            out_specs.append(pl.BlockSpec((BR, wnext_dim), lambda i: (i, 0)))
            out_shape.append(jax.ShapeDtypeStruct((NPAD, wnext_dim),
                                                  jnp.float32))

    f = pl.pallas_call(
        body,
        grid=(NB,),
        in_specs=in_specs,
        out_specs=out_specs if len(out_specs) > 1 else out_specs[0],
        out_shape=out_shape if len(out_shape) > 1 else out_shape[0],
    )
    return f


def _make_tc_pool(Wa, din, dout):
    def body(agg_ref, h_ref, wo_ref, b_ref, batch_ref,
             w0, b0, w1, b1, w2, b2, out_ref, sums_ref, cnt_ref):
        i = pl.program_id(0)
        hnew = (agg_ref[0] + agg_ref[1] + b_ref[...]) + jnp.dot(
            h_ref[...], wo_ref[...], preferred_element_type=jnp.float32)
        hnew = jnp.maximum(hnew, 0.0)
        bvec = batch_ref[0, 0]
        oh = (bvec[:, None] == lax.broadcasted_iota(jnp.int32, (1, G), 1)
              ).astype(jnp.float32)
        bs = lax.dot_general(oh, hnew, (((0,), (0,)), ((), ())),
                             preferred_element_type=jnp.float32)
        bc = lax.dot_general(oh, jnp.ones((BR, dout), jnp.float32),
                             (((0,), (0,)), ((), ())),
                             preferred_element_type=jnp.float32)

        @pl.when(i == 0)
        def _():
            sums_ref[...] = bs
            cnt_ref[...] = bc

        @pl.when(i > 0)
        def _():
            sums_ref[...] += bs
            cnt_ref[...] += bc

        @pl.when(i == NB - 1)
        def _():
            h = sums_ref[...] / jnp.maximum(cnt_ref[...], 1.0)
            h = jnp.maximum(
                jnp.dot(h, w0[...], preferred_element_type=jnp.float32)
                + b0[...], 0.0)
            h = jnp.maximum(
                jnp.dot(h, w1[...], preferred_element_type=jnp.float32)
                + b1[...], 0.0)
            out_ref[...] = jnp.dot(
                h, w2[...], preferred_element_type=jnp.float32) + b2[...]

    return pl.pallas_call(
        body,
        grid=(NB,),
        in_specs=[
            pl.BlockSpec((NC, BR, Wa), lambda i: (0, i, 0)),
            pl.BlockSpec((BR, din), lambda i: (i, 0)),
            pl.BlockSpec((din, dout), lambda i: (0, 0)),
            pl.BlockSpec((1, dout), lambda i: (0, 0)),
            pl.BlockSpec((1, 1, BR), lambda i: (i, 0, 0)),
            pl.BlockSpec((64, 64), lambda i: (0, 0)),
            pl.BlockSpec((1, 64), lambda i: (0, 0)),
            pl.BlockSpec((64, 32), lambda i: (0, 0)),
            pl.BlockSpec((1, 32), lambda i: (0, 0)),
            pl.BlockSpec((32, 1), lambda i: (0, 0)),
            pl.BlockSpec((1, 1), lambda i: (0, 0)),
        ],
        out_specs=pl.BlockSpec((G, 1), lambda i: (0, 0)),
        out_shape=jax.ShapeDtypeStruct((G, 1), jnp.float32),
        scratch_shapes=[
            pltpu.VMEM((G, dout), jnp.float32),
            pltpu.VMEM((G, dout), jnp.float32),
        ],
    )


_TC0 = _make_tc_layer(16, 16, 64, apply_wr=True)
_TC1 = _make_tc_layer(64, 64, 128, apply_wr=True, hout_split=True)
_TC2 = _make_tc_layer(128, 128, 256, apply_wr=True, agg_cat=True,
                      hin_split=True, wnext_dim=128, ynext_split=True)
_TC3 = _make_tc_layer(128, 256, 128, apply_wr=False, agg_cat=True,
                      wnext_dim=64)
_TC4 = _make_tc_pool(64, 128, 64)


def kernel(x, edge_index, edge_attr, batch,
           Wrel0, brel0, Wroot0,
           Wrel1, brel1, Wroot1,
           Wrel2, brel2, Wroot2,
           Wrel3, brel3, Wroot3,
           Wrel4, brel4, Wroot4,
           Wm0, bm0, Wm1, bm1, Wm2, bm2):
    pad = ((0, 0), (0, EPWP - EPW))
    pkf = jnp.stack([
        jnp.pad(edge_index[0].reshape(NW, EPW), pad).reshape(NW, NCH, K),
        jnp.pad(edge_index[1].reshape(NW, EPW), pad).reshape(NW, NCH, K),
    ], axis=2)
    ewf = jnp.pad(edge_attr.reshape(NW, EPW), pad).reshape(NW, NCH, K)

    padt = ((0, 0), (0, EPTP - EPT))
    pk = jnp.stack([
        jnp.pad(edge_index[0].reshape(NS, EPT), padt).reshape(NS, NCHS, K),
        jnp.pad(edge_index[1].reshape(NS, EPT), padt).reshape(NS, NCHS, K),
    ], axis=2)
    ewt = jnp.pad(edge_attr.reshape(NS, EPT), padt).reshape(NS, NCHS, K)

    xpad = jnp.zeros((NPAD, 16), jnp.float32).at[:N, :5].set(x)
    batch_pad = jnp.full((NPAD,), G, jnp.int32).at[:N].set(batch)
    batch3 = batch_pad.reshape(NB, 1, BR)

    Wr0p = jnp.zeros((16, 64), jnp.float32).at[:5].set(Wrel0)
    Wo0p = jnp.zeros((16, 64), jnp.float32).at[:5].set(Wroot0)

    b0 = brel0.reshape(1, -1)
    b1 = brel1.reshape(1, -1)
    b2 = brel2.reshape(1, -1)
    b3 = brel3.reshape(1, -1)
    b4 = brel4.reshape(1, -1)

    agg0 = _SC_SEGSUM[16](xpad, pkf, ewf)
    h1 = _TC0(agg0, xpad, Wr0p, Wo0p, b0)
    agg1 = _SC_SEGSUM[64](h1, pkf, ewf)
    h2s = _TC1(agg1, h1, Wrel1, Wroot1, b1)
    agg2 = _SC_SPLIT(h2s, pk, ewt)
    h3, y3s = _TC2(agg2, h2s, Wrel2, Wroot2, b2, Wrel3)
    agg3 = _SC_SPLIT(y3s, pk, ewt)
    h4, y4 = _TC3(agg3, h3, Wroot3, b3, Wrel4)
    agg4 = _SC_SEGSUM[64](y4, pkf, ewf)
    out = _TC4(agg4, h4, Wroot4, b4, batch3,
               Wm0, bm0.reshape(1, -1), Wm1, bm1.reshape(1, -1),
               Wm2, bm2.reshape(1, -1))
    return out
